# baseline (speedup 1.0000x reference)
import functools
import jax
import jax.numpy as jnp
from jax import lax
from jax.experimental import pallas as pl
from jax.experimental.pallas import tpu as pltpu
from jax.experimental.pallas import tpu_sc as plsc

N_NODES = 10000
N_EDGES = 320000
D_NODE = 128
D_EDGE = 16
LATENT = 32
HIDDEN = 32
NUM_OUTPUTS = 128

NC, NS = 2, 16
NW = NC * NS
NODES_PAD = 10240
EDGES_PAD = 327680
EPW = EDGES_PAD // NW
CH = 128
NCH = EPW // CH

GNBLK = EPW // (2 * CH)
SBLK = 256
SCPB = SBLK // CH
SNBLK = EPW // SBLK
NPW = NODES_PAD // NS

EROWS = EDGES_PAD // 4
NE4 = N_EDGES // 4
EBLK = 2048
NBLK = 1024

_EPS = 1e-6


def _prep_body(x_ref, wn_ref, wt_ref, aux_ref, n0_ref, sg_ref, rg_ref):
    n0 = x_ref[...] @ wn_ref[...] + aux_ref[0:1, :]
    n0_ref[...] = n0
    sg_ref[...] = n0 @ wt_ref[0:32, :]
    rg_ref[...] = n0 @ wt_ref[32:64, :]


def _prep_tables(nodes_pad, w_ne, b_ne, b_tab, c_tab):
    wt = jnp.concatenate([b_tab, c_tab], axis=0)
    aux = jnp.zeros((8, LATENT), jnp.float32).at[0, :].set(b_ne)
    grid = NODES_PAD // NBLK
    return pl.pallas_call(
        _prep_body,
        grid=(grid,),
        in_specs=[
            pl.BlockSpec((NBLK, D_NODE), lambda i: (i, 0)),
            pl.BlockSpec((D_NODE, LATENT), lambda i: (0, 0)),
            pl.BlockSpec((64, LATENT), lambda i: (0, 0)),
            pl.BlockSpec((8, LATENT), lambda i: (0, 0)),
        ],
        out_specs=[
            pl.BlockSpec((NBLK, LATENT), lambda i: (i, 0)),
            pl.BlockSpec((NBLK, LATENT), lambda i: (i, 0)),
            pl.BlockSpec((NBLK, LATENT), lambda i: (i, 0)),
        ],
        out_shape=[jax.ShapeDtypeStruct((NODES_PAD, LATENT), jnp.float32)] * 3,
    )(nodes_pad, w_ne, wt, aux)


def _edge_body(x_ref, w_ref, gs_ref, mavg_ref, aux_ref, e_ref):
    i = pl.program_id(0)
    z = x_ref[...] @ w_ref[...] + gs_ref[...] + aux_ref[0:1, :]
    mavg = mavg_ref[...]
    mu = z @ mavg
    d = z - mu
    var = (d * d) @ mavg
    h = d * lax.rsqrt(var + _EPS) * aux_ref[1:2, :] + aux_ref[2:3, :]
    e = jnp.maximum(h, 0.0)
    row = i * EBLK + lax.broadcasted_iota(jnp.int32, (EBLK, 1), 0)
    e_ref[...] = jnp.where(row < NE4, e, 0.0)


def _edge_update(x_pk, m, gsum_pk, const, scale, bias):
    k = m.shape[0]
    wbd = jax.scipy.linalg.block_diag(m, m, m, m)
    mavg = jax.scipy.linalg.block_diag(
        *([jnp.full((32, 32), 1.0 / 32, jnp.float32)] * 4))
    aux = jnp.stack([jnp.tile(const, 4), jnp.tile(scale, 4),
                     jnp.tile(bias, 4)], axis=0)
    aux = jnp.concatenate([aux, jnp.zeros((5, 128), jnp.float32)], axis=0)
    grid = EROWS // EBLK
    return pl.pallas_call(
        _edge_body,
        grid=(grid,),
        in_specs=[
            pl.BlockSpec((EBLK, 4 * k), lambda i: (i, 0)),
            pl.BlockSpec((4 * k, 128), lambda i: (0, 0)),
            pl.BlockSpec((EBLK, 128), lambda i: (i, 0)),
            pl.BlockSpec((128, 128), lambda i: (0, 0)),
            pl.BlockSpec((8, 128), lambda i: (0, 0)),
        ],
        out_specs=pl.BlockSpec((EBLK, 128), lambda i: (i, 0)),
        out_shape=jax.ShapeDtypeStruct((EROWS, 128), jnp.float32),
    )(x_pk, wbd, gsum_pk, mavg, aux)


def _node_body(n_ref, s_ref, r_ref, wc_ref, wt_ref, aux_ref,
               n1_ref, sg_ref, rg_ref, acc_ref):
    i = pl.program_id(0)

    @pl.when(i == 0)
    def _():
        acc_ref[...] = jnp.zeros_like(acc_ref)

    h = jnp.concatenate(
        [n_ref[...], s_ref[0] + s_ref[1], r_ref[0] + r_ref[1]], axis=1)
    z = h @ wc_ref[...] + aux_ref[0:1, :]
    mu = jnp.mean(z, axis=-1, keepdims=True)
    d = z - mu
    var = jnp.mean(d * d, axis=-1, keepdims=True)
    n1 = jnp.maximum(
        d * lax.rsqrt(var + _EPS) * aux_ref[1:2, :] + aux_ref[2:3, :], 0.0)
    n1_ref[...] = n1
    sg_ref[...] = n1 @ wt_ref[0:32, :]
    rg_ref[...] = n1 @ wt_ref[32:64, :]
    row = i * NBLK + lax.broadcasted_iota(jnp.int32, (NBLK, 1), 0)
    acc_ref[...] += jnp.where(row < N_NODES, n1, 0.0)


def _node_update(n_in, sent, recv, w_cat, const, scale, bias, b_tab, c_tab):
    wt = jnp.concatenate([b_tab, c_tab], axis=0)
    aux = jnp.stack([const, scale, bias], axis=0)
    aux = jnp.concatenate([aux, jnp.zeros((5, HIDDEN), jnp.float32)], axis=0)
    grid = NODES_PAD // NBLK
    bs = pl.BlockSpec((NBLK, HIDDEN), lambda i: (i, 0))
    return pl.pallas_call(
        _node_body,
        grid=(grid,),
        in_specs=[
            bs,
            pl.BlockSpec((NC, NBLK, HIDDEN), lambda i: (0, i, 0)),
            pl.BlockSpec((NC, NBLK, HIDDEN), lambda i: (0, i, 0)),
            pl.BlockSpec((3 * HIDDEN, HIDDEN), lambda i: (0, 0)),
            pl.BlockSpec((64, HIDDEN), lambda i: (0, 0)),
            pl.BlockSpec((8, HIDDEN), lambda i: (0, 0)),
        ],
        out_specs=[
            bs,
            bs,
            bs,
            pl.BlockSpec((NBLK, HIDDEN), lambda i: (0, 0)),
        ],
        out_shape=[
            jax.ShapeDtypeStruct((NODES_PAD, HIDDEN), jnp.float32),
            jax.ShapeDtypeStruct((NODES_PAD, HIDDEN), jnp.float32),
            jax.ShapeDtypeStruct((NODES_PAD, HIDDEN), jnp.float32),
            jax.ShapeDtypeStruct((NBLK, HIDDEN), jnp.float32),
        ],
    )(n_in, sent, recv, w_cat, wt, aux)


@functools.lru_cache(maxsize=None)
def _sc_mesh():
    return plsc.VectorSubcoreMesh(core_axis_name="c", subcore_axis_name="s")


_SC_PARAMS = None


def _sc_params():
    global _SC_PARAMS
    if _SC_PARAMS is None:
        _SC_PARAMS = pltpu.CompilerParams(use_tc_tiling_on_sc=False)
    return _SC_PARAMS


def _gather_sum_body(sg_hbm, rg_hbm, sidx_hbm, ridx_hbm, out_hbm,
                     idx_s, idx_r, rows_s, rows_r, rows_o, tab_s, tab_r,
                     so0, so1):
    cid = lax.axis_index("c")
    sid = lax.axis_index("s")
    wid = sid * NC + cid
    base = wid * EPW
    so = (so0, so1)

    stripe = sid * NPW

    def stage(j, _):
        off = stripe + j * CH
        pltpu.sync_copy(sg_hbm.at[pl.ds(off, CH)], rows_s)
        pltpu.sync_copy(rows_s, tab_s.at[pl.ds(off, CH)])
        pltpu.sync_copy(rg_hbm.at[pl.ds(off, CH)], rows_r)
        pltpu.sync_copy(rows_r, tab_r.at[pl.ds(off, CH)])
        return ()

    lax.fori_loop(0, NPW // CH, stage, ())
    pltpu.sync_copy(sidx_hbm.at[wid], idx_s)
    pltpu.sync_copy(ridx_hbm.at[wid], idx_r)
    plsc.subcore_barrier()

    def gather(c):
        pltpu.sync_copy(tab_s.at[idx_s.at[c]], rows_s)
        pltpu.sync_copy(tab_r.at[idx_r.at[c]], rows_r)

    def add_rows(slot):
        def add_row(r, _):
            rows_o[slot, r, pl.ds(0, 16)] = (
                rows_s[r, pl.ds(0, 16)] + rows_r[r, pl.ds(0, 16)])
            rows_o[slot, r, pl.ds(16, 16)] = (
                rows_s[r, pl.ds(16, 16)] + rows_r[r, pl.ds(16, 16)])
            return ()

        lax.fori_loop(0, CH, add_row, (), unroll=8)

    def fire_out(slot, c):
        pltpu.async_copy(rows_o.at[slot],
                         out_hbm.at[pl.ds(base + c * CH, CH)], so[slot])

    def wait_out(slot):
        pltpu.make_async_copy(rows_o.at[slot],
                              out_hbm.at[pl.ds(base, CH)], so[slot]).wait()

    for slot in (0, 1):
        gather(slot)
        add_rows(slot)
        fire_out(slot, slot)

    def body(b, _):
        for slot in (0, 1):
            c = 2 * b + slot
            gather(c)
            wait_out(slot)
            add_rows(slot)
            fire_out(slot, c)
        return ()

    lax.fori_loop(1, NCH // 2, body, ())
    for slot in (0, 1):
        wait_out(slot)


@functools.lru_cache(maxsize=None)
def _gather_sum_kernel():
    return pl.kernel(
        _gather_sum_body,
        out_type=jax.ShapeDtypeStruct((EDGES_PAD, HIDDEN), jnp.float32),
        mesh=_sc_mesh(),
        compiler_params=_sc_params(),
        scratch_types=[
            pltpu.VMEM((NCH, CH), jnp.int32),
            pltpu.VMEM((NCH, CH), jnp.int32),
            pltpu.VMEM((CH, HIDDEN), jnp.float32),
            pltpu.VMEM((CH, HIDDEN), jnp.float32),
            pltpu.VMEM((2, CH, HIDDEN), jnp.float32),
            pltpu.VMEM_SHARED((NODES_PAD, HIDDEN), jnp.float32),
            pltpu.VMEM_SHARED((NODES_PAD, HIDDEN), jnp.float32),
        ] + [pltpu.SemaphoreType.DMA] * 2,
    )


def _gather_sum(sg, rg, s_idx3, r_idx3):
    return _gather_sum_kernel()(sg, rg, s_idx3, r_idx3)


def _scatter_body(e_hbm, sidx_hbm, ridx_hbm, out_s_hbm, out_r_hbm,
                  sidx_v, ridx_v, e_v, tab_s, tab_r):
    cid = lax.axis_index("c")
    sid = lax.axis_index("s")
    wid = sid * NC + cid
    base = wid * EPW

    def zrow(r, _):
        e_v[r, pl.ds(0, 16)] = jnp.zeros((16,), jnp.float32)
        e_v[r, pl.ds(16, 16)] = jnp.zeros((16,), jnp.float32)
        return ()

    lax.fori_loop(0, SBLK, zrow, (), unroll=4)
    stripe = sid * NPW
    for tab in (tab_s, tab_r):
        pltpu.sync_copy(e_v, tab.at[pl.ds(stripe, SBLK)])
        pltpu.sync_copy(e_v.at[pl.ds(0, NPW - SBLK)],
                        tab.at[pl.ds(stripe + SBLK, NPW - SBLK)])
    plsc.subcore_barrier()

    def blk(b, _):
        off = b * SBLK
        pltpu.sync_copy(sidx_hbm.at[wid, pl.ds(b * SCPB, SCPB)], sidx_v)
        pltpu.sync_copy(ridx_hbm.at[wid, pl.ds(b * SCPB, SCPB)], ridx_v)
        pltpu.sync_copy(e_hbm.at[pl.ds(base + off, SBLK)], e_v)
        for j in range(SCPB):
            src = e_v.at[pl.ds(j * CH, CH)]
            pltpu.sync_copy(src, tab_s.at[sidx_v.at[j]], add=True)
            pltpu.sync_copy(src, tab_r.at[ridx_v.at[j]], add=True)
        return ()

    lax.fori_loop(0, SNBLK, blk, ())
    plsc.subcore_barrier()

    for tab, out in ((tab_s, out_s_hbm), (tab_r, out_r_hbm)):
        pltpu.sync_copy(tab.at[pl.ds(stripe, SBLK)], e_v)
        pltpu.sync_copy(e_v, out.at[cid, pl.ds(stripe, SBLK)])
        pltpu.sync_copy(tab.at[pl.ds(stripe + SBLK, NPW - SBLK)],
                        e_v.at[pl.ds(0, NPW - SBLK)])
        pltpu.sync_copy(e_v.at[pl.ds(0, NPW - SBLK)],
                        out.at[cid, pl.ds(stripe + SBLK, NPW - SBLK)])


@functools.lru_cache(maxsize=None)
def _scatter_segsum_kernel():
    return pl.kernel(
        _scatter_body,
        out_type=[
            jax.ShapeDtypeStruct((NC, NODES_PAD, HIDDEN), jnp.float32),
            jax.ShapeDtypeStruct((NC, NODES_PAD, HIDDEN), jnp.float32),
        ],
        mesh=_sc_mesh(),
        compiler_params=_sc_params(),
        scratch_types=[
            pltpu.VMEM((SCPB, CH), jnp.int32),
            pltpu.VMEM((SCPB, CH), jnp.int32),
            pltpu.VMEM((SBLK, HIDDEN), jnp.float32),
            pltpu.VMEM_SHARED((NODES_PAD, HIDDEN), jnp.float32),
            pltpu.VMEM_SHARED((NODES_PAD, HIDDEN), jnp.float32),
        ],
    )


def _scatter_segsum(e_lin, s_idx3, r_idx3):
    return _scatter_segsum_kernel()(e_lin, s_idx3, r_idx3)


def _split_edge_w(w):
    return (w[0:LATENT], w[LATENT:2 * LATENT], w[2 * LATENT:3 * LATENT],
            w[3 * LATENT:])


def _ln_relu_vec(z, scale, bias):
    mu = jnp.mean(z)
    var = jnp.mean(jnp.square(z - mu))
    return jnp.maximum((z - mu) * lax.rsqrt(var + _EPS) * scale + bias, 0.0)


def kernel(nodes, edges, senders, receivers, train, params):
    del train
    f32 = jnp.float32
    nodes_pad = jnp.zeros((NODES_PAD, D_NODE), f32).at[:N_NODES].set(nodes)
    edges_pk = jnp.zeros((EDGES_PAD, D_EDGE), f32).at[:N_EDGES].set(edges)
    edges_pk = edges_pk.reshape(EROWS, 4 * D_EDGE)
    s_idx3 = (jnp.zeros((EDGES_PAD,), jnp.int32).at[:N_EDGES].set(senders)
              .reshape(NW, NCH, CH))
    r_idx3 = (jnp.zeros((EDGES_PAD,), jnp.int32).at[:N_EDGES].set(receivers)
              .reshape(NW, NCH, CH))

    p = params
    st1, st2 = p["steps"][0], p["steps"][1]

    a1, b1, c1, _ = _split_edge_w(st1["edge"]["dense0"]["w"])
    m1 = p["edge_embed"]["w"] @ a1
    ce1 = p["edge_embed"]["b"] @ a1 + st1["edge"]["dense0"]["b"]

    n0, sg1, rg1 = _prep_tables(nodes_pad, p["node_embed"]["w"],
                                p["node_embed"]["b"], b1, c1)

    gsum1 = _gather_sum(sg1, rg1, s_idx3, r_idx3).reshape(EROWS, 128)
    e1_pk = _edge_update(edges_pk, m1, gsum1, ce1,
                         st1["edge"]["ln0"]["scale"],
                         st1["edge"]["ln0"]["bias"])

    sent1, recv1 = _scatter_segsum(
        e1_pk.reshape(EDGES_PAD, HIDDEN), s_idx3, r_idx3)
    e_agg1 = jnp.sum(e1_pk.reshape(EDGES_PAD, HIDDEN), axis=0)

    a2, b2, c2, g2w = _split_edge_w(st2["edge"]["dense0"]["w"])
    n1, sg2, rg2, acc_n1 = _node_update(
        n0, sent1, recv1, st1["node"]["dense0"]["w"][:3 * HIDDEN],
        st1["node"]["dense0"]["b"], st1["node"]["ln0"]["scale"],
        st1["node"]["ln0"]["bias"], b2, c2)
    n_agg1 = jnp.sum(acc_n1, axis=0)

    g0 = jnp.zeros((NUM_OUTPUTS,), f32)
    gin1 = jnp.concatenate([n_agg1, e_agg1, g0])
    gz1 = gin1 @ st1["global"]["dense0"]["w"] + st1["global"]["dense0"]["b"]
    g1 = _ln_relu_vec(gz1, st1["global"]["ln0"]["scale"],
                      st1["global"]["ln0"]["bias"])

    ce2 = g1 @ g2w + st2["edge"]["dense0"]["b"]
    gsum2 = _gather_sum(sg2, rg2, s_idx3, r_idx3).reshape(EROWS, 128)
    e2_pk = _edge_update(e1_pk, a2, gsum2, ce2,
                         st2["edge"]["ln0"]["scale"],
                         st2["edge"]["ln0"]["bias"])

    sent2, recv2 = _scatter_segsum(
        e2_pk.reshape(EDGES_PAD, HIDDEN), s_idx3, r_idx3)
    e_agg2 = jnp.sum(e2_pk.reshape(EDGES_PAD, HIDDEN), axis=0)

    wn2 = st2["node"]["dense0"]["w"]
    cn2 = g1 @ wn2[3 * HIDDEN:] + st2["node"]["dense0"]["b"]
    _, _, _, acc_n2 = _node_update(
        n1, sent2, recv2, wn2[:3 * HIDDEN], cn2,
        st2["node"]["ln0"]["scale"], st2["node"]["ln0"]["bias"], b2, c2)
    n_agg2 = jnp.sum(acc_n2, axis=0)

    gin2 = jnp.concatenate([n_agg2, e_agg2, g1])
    gz2 = gin2 @ st2["global"]["dense0"]["w"] + st2["global"]["dense0"]["b"]
    g2 = _ln_relu_vec(gz2, st2["global"]["ln0"]["scale"],
                      st2["global"]["ln0"]["bias"])

    out = g2 @ p["decoder"]["w"] + p["decoder"]["b"]
    return out.reshape(1, NUM_OUTPUTS)


# speedup vs baseline: 1.0684x; 1.0684x over previous
import functools
import jax
import jax.numpy as jnp
from jax import lax
from jax.experimental import pallas as pl
from jax.experimental.pallas import tpu as pltpu
from jax.experimental.pallas import tpu_sc as plsc

N_NODES = 10000
N_EDGES = 320000
D_NODE = 128
D_EDGE = 16
LATENT = 32
HIDDEN = 32
NUM_OUTPUTS = 128

NC, NS = 2, 16
NW = NC * NS
NODES_PAD = 10240
EDGES_PAD = 327680
EPW = EDGES_PAD // NW
CH = 128
NCH = EPW // CH

GNBLK = EPW // (2 * CH)
SBLK = 256
SCPB = SBLK // CH
SNBLK = EPW // SBLK
NPW = NODES_PAD // NS

EROWS = EDGES_PAD // 4
NE4 = N_EDGES // 4
EBLK = 2048
NBLK = 1024

_EPS = 1e-6


def _prep_body(x_ref, wn_ref, wt_ref, aux_ref, n0_ref, sg_ref, rg_ref):
    n0 = x_ref[...] @ wn_ref[...] + aux_ref[0:1, :]
    n0_ref[...] = n0
    sg_ref[...] = n0 @ wt_ref[0:32, :]
    rg_ref[...] = n0 @ wt_ref[32:64, :]


def _prep_tables(nodes_pad, w_ne, b_ne, b_tab, c_tab):
    wt = jnp.concatenate([b_tab, c_tab], axis=0)
    aux = jnp.zeros((8, LATENT), jnp.float32).at[0, :].set(b_ne)
    grid = NODES_PAD // NBLK
    return pl.pallas_call(
        _prep_body,
        grid=(grid,),
        in_specs=[
            pl.BlockSpec((NBLK, D_NODE), lambda i: (i, 0)),
            pl.BlockSpec((D_NODE, LATENT), lambda i: (0, 0)),
            pl.BlockSpec((64, LATENT), lambda i: (0, 0)),
            pl.BlockSpec((8, LATENT), lambda i: (0, 0)),
        ],
        out_specs=[
            pl.BlockSpec((NBLK, LATENT), lambda i: (i, 0)),
            pl.BlockSpec((NBLK, LATENT), lambda i: (i, 0)),
            pl.BlockSpec((NBLK, LATENT), lambda i: (i, 0)),
        ],
        out_shape=[jax.ShapeDtypeStruct((NODES_PAD, LATENT), jnp.float32)] * 3,
    )(nodes_pad, w_ne, wt, aux)


def _edge_body(x_ref, w_ref, gs_ref, mavg_ref, aux_ref, e_ref, acc_ref):
    i = pl.program_id(0)

    @pl.when(i == 0)
    def _():
        acc_ref[...] = jnp.zeros_like(acc_ref)

    z = x_ref[...] @ w_ref[...] + gs_ref[...] + aux_ref[0:1, :]
    mavg = mavg_ref[...]
    mu = z @ mavg
    d = z - mu
    var = (d * d) @ mavg
    h = d * lax.rsqrt(var + _EPS) * aux_ref[1:2, :] + aux_ref[2:3, :]
    e = jnp.maximum(h, 0.0)
    row = i * EBLK + lax.broadcasted_iota(jnp.int32, (EBLK, 1), 0)
    e = jnp.where(row < NE4, e, 0.0)
    e_ref[...] = e
    acc_ref[...] += e


def _edge_update(x_pk, m, gsum_pk, const, scale, bias):
    k = m.shape[0]
    wbd = jax.scipy.linalg.block_diag(m, m, m, m)
    mavg = jax.scipy.linalg.block_diag(
        *([jnp.full((32, 32), 1.0 / 32, jnp.float32)] * 4))
    aux = jnp.stack([jnp.tile(const, 4), jnp.tile(scale, 4),
                     jnp.tile(bias, 4)], axis=0)
    aux = jnp.concatenate([aux, jnp.zeros((5, 128), jnp.float32)], axis=0)
    grid = EROWS // EBLK
    return pl.pallas_call(
        _edge_body,
        grid=(grid,),
        in_specs=[
            pl.BlockSpec((EBLK, 4 * k), lambda i: (i, 0)),
            pl.BlockSpec((4 * k, 128), lambda i: (0, 0)),
            pl.BlockSpec((EBLK, 128), lambda i: (i, 0)),
            pl.BlockSpec((128, 128), lambda i: (0, 0)),
            pl.BlockSpec((8, 128), lambda i: (0, 0)),
        ],
        out_specs=[
            pl.BlockSpec((EBLK, 128), lambda i: (i, 0)),
            pl.BlockSpec((EBLK, 128), lambda i: (0, 0)),
        ],
        out_shape=[
            jax.ShapeDtypeStruct((EROWS, 128), jnp.float32),
            jax.ShapeDtypeStruct((EBLK, 128), jnp.float32),
        ],
    )(x_pk, wbd, gsum_pk, mavg, aux)


def _node_body(n_ref, s_ref, r_ref, wc_ref, wt_ref, aux_ref,
               n1_ref, sg_ref, rg_ref, acc_ref):
    i = pl.program_id(0)

    @pl.when(i == 0)
    def _():
        acc_ref[...] = jnp.zeros_like(acc_ref)

    h = jnp.concatenate(
        [n_ref[...], s_ref[0] + s_ref[1], r_ref[0] + r_ref[1]], axis=1)
    z = h @ wc_ref[...] + aux_ref[0:1, :]
    mu = jnp.mean(z, axis=-1, keepdims=True)
    d = z - mu
    var = jnp.mean(d * d, axis=-1, keepdims=True)
    n1 = jnp.maximum(
        d * lax.rsqrt(var + _EPS) * aux_ref[1:2, :] + aux_ref[2:3, :], 0.0)
    n1_ref[...] = n1
    sg_ref[...] = n1 @ wt_ref[0:32, :]
    rg_ref[...] = n1 @ wt_ref[32:64, :]
    row = i * NBLK + lax.broadcasted_iota(jnp.int32, (NBLK, 1), 0)
    acc_ref[...] += jnp.where(row < N_NODES, n1, 0.0)


def _node_update(n_in, sent, recv, w_cat, const, scale, bias, b_tab, c_tab):
    wt = jnp.concatenate([b_tab, c_tab], axis=0)
    aux = jnp.stack([const, scale, bias], axis=0)
    aux = jnp.concatenate([aux, jnp.zeros((5, HIDDEN), jnp.float32)], axis=0)
    grid = NODES_PAD // NBLK
    bs = pl.BlockSpec((NBLK, HIDDEN), lambda i: (i, 0))
    return pl.pallas_call(
        _node_body,
        grid=(grid,),
        in_specs=[
            bs,
            pl.BlockSpec((NC, NBLK, HIDDEN), lambda i: (0, i, 0)),
            pl.BlockSpec((NC, NBLK, HIDDEN), lambda i: (0, i, 0)),
            pl.BlockSpec((3 * HIDDEN, HIDDEN), lambda i: (0, 0)),
            pl.BlockSpec((64, HIDDEN), lambda i: (0, 0)),
            pl.BlockSpec((8, HIDDEN), lambda i: (0, 0)),
        ],
        out_specs=[
            bs,
            bs,
            bs,
            pl.BlockSpec((NBLK, HIDDEN), lambda i: (0, 0)),
        ],
        out_shape=[
            jax.ShapeDtypeStruct((NODES_PAD, HIDDEN), jnp.float32),
            jax.ShapeDtypeStruct((NODES_PAD, HIDDEN), jnp.float32),
            jax.ShapeDtypeStruct((NODES_PAD, HIDDEN), jnp.float32),
            jax.ShapeDtypeStruct((NBLK, HIDDEN), jnp.float32),
        ],
    )(n_in, sent, recv, w_cat, wt, aux)


@functools.lru_cache(maxsize=None)
def _sc_mesh():
    return plsc.VectorSubcoreMesh(core_axis_name="c", subcore_axis_name="s")


_SC_PARAMS = None


def _sc_params():
    global _SC_PARAMS
    if _SC_PARAMS is None:
        _SC_PARAMS = pltpu.CompilerParams(use_tc_tiling_on_sc=False)
    return _SC_PARAMS


def _gather_sum_body(sg_hbm, rg_hbm, sidx_hbm, ridx_hbm, out_hbm,
                     idx_s, idx_r, rows_s, rows_r, rows_o, tab_s, tab_r,
                     so0, so1):
    cid = lax.axis_index("c")
    sid = lax.axis_index("s")
    wid = sid * NC + cid
    base = wid * EPW
    so = (so0, so1)

    stripe = sid * NPW

    def stage(j, _):
        off = stripe + j * CH
        pltpu.sync_copy(sg_hbm.at[pl.ds(off, CH)], rows_s)
        pltpu.sync_copy(rows_s, tab_s.at[pl.ds(off, CH)])
        pltpu.sync_copy(rg_hbm.at[pl.ds(off, CH)], rows_r)
        pltpu.sync_copy(rows_r, tab_r.at[pl.ds(off, CH)])
        return ()

    lax.fori_loop(0, NPW // CH, stage, ())
    pltpu.sync_copy(sidx_hbm.at[wid], idx_s)
    pltpu.sync_copy(ridx_hbm.at[wid], idx_r)
    plsc.subcore_barrier()

    def gather(c):
        pltpu.sync_copy(tab_s.at[idx_s.at[c]], rows_s)
        pltpu.sync_copy(tab_r.at[idx_r.at[c]], rows_r)

    def add_rows(slot):
        def add_row(r, _):
            rows_o[slot, r, pl.ds(0, 16)] = (
                rows_s[r, pl.ds(0, 16)] + rows_r[r, pl.ds(0, 16)])
            rows_o[slot, r, pl.ds(16, 16)] = (
                rows_s[r, pl.ds(16, 16)] + rows_r[r, pl.ds(16, 16)])
            return ()

        lax.fori_loop(0, CH, add_row, (), unroll=8)

    def fire_out(slot, c):
        pltpu.async_copy(rows_o.at[slot],
                         out_hbm.at[pl.ds(base + c * CH, CH)], so[slot])

    def wait_out(slot):
        pltpu.make_async_copy(rows_o.at[slot],
                              out_hbm.at[pl.ds(base, CH)], so[slot]).wait()

    for slot in (0, 1):
        gather(slot)
        add_rows(slot)
        fire_out(slot, slot)

    def body(b, _):
        for slot in (0, 1):
            c = 2 * b + slot
            gather(c)
            wait_out(slot)
            add_rows(slot)
            fire_out(slot, c)
        return ()

    lax.fori_loop(1, NCH // 2, body, ())
    for slot in (0, 1):
        wait_out(slot)


@functools.lru_cache(maxsize=None)
def _gather_sum_kernel():
    return pl.kernel(
        _gather_sum_body,
        out_type=jax.ShapeDtypeStruct((EDGES_PAD, HIDDEN), jnp.float32),
        mesh=_sc_mesh(),
        compiler_params=_sc_params(),
        scratch_types=[
            pltpu.VMEM((NCH, CH), jnp.int32),
            pltpu.VMEM((NCH, CH), jnp.int32),
            pltpu.VMEM((CH, HIDDEN), jnp.float32),
            pltpu.VMEM((CH, HIDDEN), jnp.float32),
            pltpu.VMEM((2, CH, HIDDEN), jnp.float32),
            pltpu.VMEM_SHARED((NODES_PAD, HIDDEN), jnp.float32),
            pltpu.VMEM_SHARED((NODES_PAD, HIDDEN), jnp.float32),
        ] + [pltpu.SemaphoreType.DMA] * 2,
    )


def _gather_sum(sg, rg, s_idx3, r_idx3):
    return _gather_sum_kernel()(sg, rg, s_idx3, r_idx3)


def _scatter_body(e_hbm, sidx_hbm, ridx_hbm, out_s_hbm, out_r_hbm,
                  sidx_v, ridx_v, e_v, tab_s, tab_r):
    cid = lax.axis_index("c")
    sid = lax.axis_index("s")
    wid = sid * NC + cid
    base = wid * EPW

    def zrow(r, _):
        e_v[r, pl.ds(0, 16)] = jnp.zeros((16,), jnp.float32)
        e_v[r, pl.ds(16, 16)] = jnp.zeros((16,), jnp.float32)
        return ()

    lax.fori_loop(0, SBLK, zrow, (), unroll=4)
    stripe = sid * NPW
    for tab in (tab_s, tab_r):
        pltpu.sync_copy(e_v, tab.at[pl.ds(stripe, SBLK)])
        pltpu.sync_copy(e_v.at[pl.ds(0, NPW - SBLK)],
                        tab.at[pl.ds(stripe + SBLK, NPW - SBLK)])
    plsc.subcore_barrier()

    def blk(b, _):
        off = b * SBLK
        pltpu.sync_copy(sidx_hbm.at[wid, pl.ds(b * SCPB, SCPB)], sidx_v)
        pltpu.sync_copy(ridx_hbm.at[wid, pl.ds(b * SCPB, SCPB)], ridx_v)
        pltpu.sync_copy(e_hbm.at[pl.ds(base + off, SBLK)], e_v)
        for j in range(SCPB):
            src = e_v.at[pl.ds(j * CH, CH)]
            pltpu.sync_copy(src, tab_s.at[sidx_v.at[j]], add=True)
            pltpu.sync_copy(src, tab_r.at[ridx_v.at[j]], add=True)
        return ()

    lax.fori_loop(0, SNBLK, blk, ())
    plsc.subcore_barrier()

    for tab, out in ((tab_s, out_s_hbm), (tab_r, out_r_hbm)):
        pltpu.sync_copy(tab.at[pl.ds(stripe, SBLK)], e_v)
        pltpu.sync_copy(e_v, out.at[cid, pl.ds(stripe, SBLK)])
        pltpu.sync_copy(tab.at[pl.ds(stripe + SBLK, NPW - SBLK)],
                        e_v.at[pl.ds(0, NPW - SBLK)])
        pltpu.sync_copy(e_v.at[pl.ds(0, NPW - SBLK)],
                        out.at[cid, pl.ds(stripe + SBLK, NPW - SBLK)])


@functools.lru_cache(maxsize=None)
def _scatter_segsum_kernel():
    return pl.kernel(
        _scatter_body,
        out_type=[
            jax.ShapeDtypeStruct((NC, NODES_PAD, HIDDEN), jnp.float32),
            jax.ShapeDtypeStruct((NC, NODES_PAD, HIDDEN), jnp.float32),
        ],
        mesh=_sc_mesh(),
        compiler_params=_sc_params(),
        scratch_types=[
            pltpu.VMEM((SCPB, CH), jnp.int32),
            pltpu.VMEM((SCPB, CH), jnp.int32),
            pltpu.VMEM((SBLK, HIDDEN), jnp.float32),
            pltpu.VMEM_SHARED((NODES_PAD, HIDDEN), jnp.float32),
            pltpu.VMEM_SHARED((NODES_PAD, HIDDEN), jnp.float32),
        ],
    )


def _scatter_segsum(e_lin, s_idx3, r_idx3):
    return _scatter_segsum_kernel()(e_lin, s_idx3, r_idx3)


def _split_edge_w(w):
    return (w[0:LATENT], w[LATENT:2 * LATENT], w[2 * LATENT:3 * LATENT],
            w[3 * LATENT:])


def _ln_relu_vec(z, scale, bias):
    mu = jnp.mean(z)
    var = jnp.mean(jnp.square(z - mu))
    return jnp.maximum((z - mu) * lax.rsqrt(var + _EPS) * scale + bias, 0.0)


def kernel(nodes, edges, senders, receivers, train, params):
    del train
    f32 = jnp.float32
    nodes_pad = jnp.zeros((NODES_PAD, D_NODE), f32).at[:N_NODES].set(nodes)
    edges_pk = jnp.zeros((EDGES_PAD, D_EDGE), f32).at[:N_EDGES].set(edges)
    edges_pk = edges_pk.reshape(EROWS, 4 * D_EDGE)
    s_idx3 = (jnp.zeros((EDGES_PAD,), jnp.int32).at[:N_EDGES].set(senders)
              .reshape(NW, NCH, CH))
    r_idx3 = (jnp.zeros((EDGES_PAD,), jnp.int32).at[:N_EDGES].set(receivers)
              .reshape(NW, NCH, CH))

    p = params
    st1, st2 = p["steps"][0], p["steps"][1]

    a1, b1, c1, _ = _split_edge_w(st1["edge"]["dense0"]["w"])
    m1 = p["edge_embed"]["w"] @ a1
    ce1 = p["edge_embed"]["b"] @ a1 + st1["edge"]["dense0"]["b"]

    n0, sg1, rg1 = _prep_tables(nodes_pad, p["node_embed"]["w"],
                                p["node_embed"]["b"], b1, c1)

    gsum1 = _gather_sum(sg1, rg1, s_idx3, r_idx3).reshape(EROWS, 128)
    e1_pk, acc_e1 = _edge_update(edges_pk, m1, gsum1, ce1,
                                 st1["edge"]["ln0"]["scale"],
                                 st1["edge"]["ln0"]["bias"])
    e_agg1 = jnp.sum(acc_e1, axis=0).reshape(4, HIDDEN).sum(axis=0)

    sent1, recv1 = _scatter_segsum(
        e1_pk.reshape(EDGES_PAD, HIDDEN), s_idx3, r_idx3)

    a2, b2, c2, g2w = _split_edge_w(st2["edge"]["dense0"]["w"])
    n1, sg2, rg2, acc_n1 = _node_update(
        n0, sent1, recv1, st1["node"]["dense0"]["w"][:3 * HIDDEN],
        st1["node"]["dense0"]["b"], st1["node"]["ln0"]["scale"],
        st1["node"]["ln0"]["bias"], b2, c2)
    n_agg1 = jnp.sum(acc_n1, axis=0)

    g0 = jnp.zeros((NUM_OUTPUTS,), f32)
    gin1 = jnp.concatenate([n_agg1, e_agg1, g0])
    gz1 = gin1 @ st1["global"]["dense0"]["w"] + st1["global"]["dense0"]["b"]
    g1 = _ln_relu_vec(gz1, st1["global"]["ln0"]["scale"],
                      st1["global"]["ln0"]["bias"])

    ce2 = g1 @ g2w + st2["edge"]["dense0"]["b"]
    gsum2 = _gather_sum(sg2, rg2, s_idx3, r_idx3).reshape(EROWS, 128)
    e2_pk, acc_e2 = _edge_update(e1_pk, a2, gsum2, ce2,
                                 st2["edge"]["ln0"]["scale"],
                                 st2["edge"]["ln0"]["bias"])
    e_agg2 = jnp.sum(acc_e2, axis=0).reshape(4, HIDDEN).sum(axis=0)

    sent2, recv2 = _scatter_segsum(
        e2_pk.reshape(EDGES_PAD, HIDDEN), s_idx3, r_idx3)

    wn2 = st2["node"]["dense0"]["w"]
    cn2 = g1 @ wn2[3 * HIDDEN:] + st2["node"]["dense0"]["b"]
    _, _, _, acc_n2 = _node_update(
        n1, sent2, recv2, wn2[:3 * HIDDEN], cn2,
        st2["node"]["ln0"]["scale"], st2["node"]["ln0"]["bias"], b2, c2)
    n_agg2 = jnp.sum(acc_n2, axis=0)

    gin2 = jnp.concatenate([n_agg2, e_agg2, g1])
    gz2 = gin2 @ st2["global"]["dense0"]["w"] + st2["global"]["dense0"]["b"]
    g2 = _ln_relu_vec(gz2, st2["global"]["ln0"]["scale"],
                      st2["global"]["ln0"]["bias"])

    out = g2 @ p["decoder"]["w"] + p["decoder"]["b"]
    return out.reshape(1, NUM_OUTPUTS)
